# Initial kernel scaffold; baseline (speedup 1.0000x reference)
import jax
import jax.numpy as jnp
from jax.experimental import pallas as pl


def kernel(num_nodes, z, edge_index, batch, z_table, W0, b0, W1, b1, W2, b2, W3, b3, c1w, c1b, c2w, c2b, m1w, m1b, m2w, m2b):
    raise NotImplementedError("write your pallas kernel here")


# speedup vs baseline: 1.0000x; 1.0000x over previous
import jax
import jax.numpy as jnp
from jax.experimental import pallas as pl

B_GRAPHS = 256
K = 30


def _gcn_conv(x, edge_index, W, b, n):
    src = jnp.concatenate([edge_index[0], jnp.arange(n)])
    dst = jnp.concatenate([edge_index[1], jnp.arange(n)])
    deg = jnp.zeros((n,), x.dtype).at[dst].add(1.0)
    dinv = jnp.where(deg > 0, deg ** -0.5, 0.0)
    norm = dinv[src] * dinv[dst]
    h = x @ W
    out = jnp.zeros((n, W.shape[1]), x.dtype).at[dst].add(h[src] * norm[:, None])
    return out + b


def _conv1d(x, w, b, stride):
    out = jax.lax.conv_general_dilated(
        x, w, (stride,), 'VALID', dimension_numbers=('NCH', 'OIH', 'NCH'))
    return out + b[None, :, None]


def _head_body(h_ref, w1_ref, b1_ref, w2_ref, b2_ref, o_ref):
    h = h_ref[...]
    a = jnp.maximum(jnp.dot(h, w1_ref[...],
                            preferred_element_type=jnp.float32) + b1_ref[...][None, :], 0.0)
    o_ref[...] = jnp.dot(a, w2_ref[...],
                         preferred_element_type=jnp.float32) + b2_ref[...][None, :]


def kernel(num_nodes, z, edge_index, batch, z_table, W0, b0, W1, b1, W2, b2,
           W3, b3, c1w, c1b, c2w, c2b, m1w, m1b, m2w, m2b):
    n = z.shape[0]
    x = z_table[z]
    xs = []
    h = x
    for (W, b) in [(W0, b0), (W1, b1), (W2, b2), (W3, b3)]:
        h = jnp.tanh(_gcn_conv(h, edge_index, W, b, n))
        xs.append(h)
    x = jnp.concatenate(xs, axis=-1)
    D = x.shape[1]
    order = jnp.lexsort((-x[:, -1], batch))
    xs_ = x[order]
    bs_ = batch[order]
    start = jnp.searchsorted(bs_, jnp.arange(B_GRAPHS))
    pos = jnp.arange(n) + (num_nodes - n) - start[bs_]
    valid = pos < K
    dense = jnp.zeros((B_GRAPHS, K, D), x.dtype).at[
        bs_, jnp.where(valid, pos, 0)
    ].add(jnp.where(valid[:, None], xs_, 0.0))
    xf = dense.reshape(B_GRAPHS, 1, K * D)
    h = jax.nn.relu(_conv1d(xf, c1w, c1b, D))
    h = jax.lax.reduce_window(h, -jnp.inf, jax.lax.max, (1, 1, 2), (1, 1, 2), 'VALID')
    h = jax.nn.relu(_conv1d(h, c2w, c2b, 1))
    h = h.reshape(B_GRAPHS, -1)
    out = pl.pallas_call(
        _head_body,
        out_shape=jax.ShapeDtypeStruct((B_GRAPHS, 1), jnp.float32),
    )(h, m1w, m1b, m2w, m2b)
    return out


# speedup vs baseline: 1.7172x; 1.7172x over previous
import functools

import jax
import jax.numpy as jnp
from jax import lax
from jax.experimental import pallas as pl
from jax.experimental.pallas import tpu as pltpu
from jax.experimental.pallas import tpu_sc as plsc

N = 100000
E = 1600000
ROWS = 12544
E_PAD = ROWS * 128
NACC = 100352
STRIPE = NACC // 16
B_GRAPHS = 256
K = 30

_MESH = plsc.VectorSubcoreMesh(core_axis_name="c", subcore_axis_name="s")
_CP = pltpu.CompilerParams(use_tc_tiling_on_sc=False, needs_layout_passes=False)


def _scal_body(v_hbm, srcoff_hbm, dst_hbm, zeros_hbm, out_hbm,
               idx_s, idx_d, vals, acc, sem):
    c = lax.axis_index("c")
    s = lax.axis_index("s")
    pltpu.sync_copy(zeros_hbm, acc.at[pl.ds(s * STRIPE, STRIPE)])
    plsc.subcore_barrier()

    def body(i, carry):
        base = (c * 16 + s) * 392 + i * 4
        pltpu.sync_copy(srcoff_hbm.at[0, pl.ds(base, 4)], idx_s)
        pltpu.sync_copy(dst_hbm.at[pl.ds(base, 4)], idx_d)
        for j in range(4):
            pltpu.async_copy(v_hbm.at[idx_s.at[j]], vals.at[j], sem).wait()
        for j in range(4):
            pltpu.sync_copy(vals.at[j], acc.at[idx_d.at[j]], add=True)
        return carry

    lax.fori_loop(0, 98, body, 0)
    plsc.subcore_barrier()
    pltpu.sync_copy(acc.at[pl.ds(s * STRIPE, STRIPE)],
                    out_hbm.at[c, pl.ds(s * STRIPE, STRIPE)])


_sc_scal = functools.partial(
    pl.kernel,
    _scal_body,
    out_type=jax.ShapeDtypeStruct((2, NACC), jnp.float32),
    mesh=_MESH,
    compiler_params=_CP,
    scratch_types=[
        pltpu.VMEM((4, 128), jnp.int32),
        pltpu.VMEM((4, 128), jnp.int32),
        pltpu.VMEM((4, 128), jnp.float32),
        pltpu.VMEM_SHARED((NACC,), jnp.float32),
        pltpu.SemaphoreType.DMA,
    ],
)()


def _norm_body(dinv_hbm, srcoff_hbm, dst_hbm, norm_hbm, dv, idx_s, idx_d, nb):
    c = lax.axis_index("c")
    s = lax.axis_index("s")
    pltpu.sync_copy(dinv_hbm, dv)

    def body(i, carry):
        base = (c * 16 + s) * 392 + i * 4
        pltpu.sync_copy(srcoff_hbm.at[0, pl.ds(base, 4)], idx_s)
        pltpu.sync_copy(dst_hbm.at[pl.ds(base, 4)], idx_d)
        for j in range(4):
            for g in range(8):
                s16 = idx_s[j, pl.ds(g * 16, 16)]
                d16 = idx_d[j, pl.ds(g * 16, 16)]
                a = plsc.load_gather(dv, [s16])
                b = plsc.load_gather(dv, [d16])
                nb[j, pl.ds(g * 16, 16)] = a * b
        pltpu.sync_copy(nb, norm_hbm.at[pl.ds(base, 4)])
        return carry

    lax.fori_loop(0, 98, body, 0)


_sc_norm = functools.partial(
    pl.kernel,
    _norm_body,
    out_type=jax.ShapeDtypeStruct((ROWS, 128), jnp.float32),
    mesh=_MESH,
    compiler_params=_CP,
    scratch_types=[
        pltpu.VMEM((NACC,), jnp.float32),
        pltpu.VMEM((4, 128), jnp.int32),
        pltpu.VMEM((4, 128), jnp.int32),
        pltpu.VMEM((4, 128), jnp.float32),
    ],
)()


def _upd_body(q_hbm, norm_hbm, srcoff_hbm, rowse_hbm,
              idx_s, nb, rows, sem):
    c = lax.axis_index("c")
    s = lax.axis_index("s")
    iota = lax.iota(jnp.int32, 16)

    def body(i, carry):
        base = s * 784 + i * 4
        pltpu.sync_copy(srcoff_hbm.at[c, pl.ds(base, 4)], idx_s)
        d0 = pltpu.async_copy(q_hbm.at[idx_s.at[0]], rows.at[0], sem)
        d1 = pltpu.async_copy(q_hbm.at[idx_s.at[1]], rows.at[1], sem)
        d2 = pltpu.async_copy(q_hbm.at[idx_s.at[2]], rows.at[2], sem)
        d3 = pltpu.async_copy(q_hbm.at[idx_s.at[3]], rows.at[3], sem)
        pltpu.sync_copy(norm_hbm.at[pl.ds(base, 4)], nb)
        d0.wait(); d1.wait(); d2.wait(); d3.wait()
        for j in range(4):
            jc = jnp.full((16,), j, jnp.int32)
            for g in range(8):
                n16 = nb[j, pl.ds(g * 16, 16)]
                e16 = g * 16 + iota
                for col in range(16):
                    c16 = jnp.full((16,), col, jnp.int32)
                    vv = plsc.load_gather(rows, [jc, e16, c16])
                    plsc.store_scatter(rows, [jc, e16, c16], vv * n16)
        pltpu.sync_copy(rows, rowse_hbm.at[c, pl.ds(base, 4)])
        return carry

    lax.fori_loop(0, 196, body, 0)


_sc_upd = functools.partial(
    pl.kernel,
    _upd_body,
    out_type=jax.ShapeDtypeStruct((2, ROWS, 128, 16), jnp.float32),
    mesh=_MESH,
    compiler_params=_CP,
    scratch_types=[
        pltpu.VMEM((4, 128), jnp.int32),
        pltpu.VMEM((4, 128), jnp.float32),
        pltpu.VMEM((4, 128, 16), jnp.float32),
        pltpu.SemaphoreType.DMA,
    ],
)()


def _sortpool_body(v_hbm, starts_hbm, ends_hbm, h0_hbm, h1_hbm, h2_hbm,
                   d0_hbm, d1_hbm, d2_hbm, dv_hbm,
                   vbuf, sbuf, ebuf, nids, rows, vals, sem):
    c = lax.axis_index("c")
    s = lax.axis_index("s")
    w = c * 16 + s
    pltpu.sync_copy(v_hbm, vbuf)
    pltpu.sync_copy(starts_hbm, sbuf)
    pltpu.sync_copy(ends_hbm, ebuf)
    iota = lax.iota(jnp.int32, 16)
    lane0 = iota == 0
    big = jnp.full((16,), 1 << 29, jnp.int32)

    def per_graph(gi, carry):
        g = w * 8 + gi
        off = (g // 16) * 16
        lane = g - off
        svec = jnp.where(iota == lane, sbuf[pl.ds(off, 16)], 0)
        evec = jnp.where(iota == lane, ebuf[pl.ds(off, 16)], 0)
        sg = jnp.max(svec)
        eg = jnp.max(evec)
        cs = sg // 16
        nch = (eg + 15) // 16 - cs
        cnt = eg - sg
        nids[pl.ds(0, 16)] = jnp.full((16,), N, jnp.int32)
        nids[pl.ds(16, 16)] = jnp.full((16,), N, jnp.int32)

        def per_round(r, carry2):
            def per_chunk(i, mp):
                m16, p16 = mp
                base = (cs + i) * 16
                val = vbuf[pl.ds(base, 16)]
                posv = base + iota
                ok = (posv >= sg) & (posv < eg)
                vv = jnp.where(ok, val, jnp.float32(-4.0))
                upd = vv > m16
                return (jnp.where(upd, vv, m16), jnp.where(upd, posv, p16))

            m16, p16 = lax.fori_loop(
                0, nch, per_chunk,
                (jnp.full((16,), -4.0, jnp.float32), big))
            M = jnp.max(m16)
            pm = jnp.where(m16 == M, p16, big)
            P = jnp.min(pm)
            psplat = jnp.full((16,), P, jnp.int32)
            plsc.store_scatter(nids, [jnp.full((16,), r, jnp.int32)],
                               psplat, mask=lane0)
            plsc.store_scatter(vbuf, [psplat],
                               jnp.full((16,), -4.0, jnp.float32), mask=lane0)
            return carry2

        lax.fori_loop(0, jnp.minimum(cnt, K), per_round, 0)

        pltpu.async_copy(h0_hbm.at[nids], rows, sem).wait()
        pltpu.sync_copy(rows.at[pl.ds(0, K)], d0_hbm.at[g])
        pltpu.async_copy(h1_hbm.at[nids], rows, sem).wait()
        pltpu.sync_copy(rows.at[pl.ds(0, K)], d1_hbm.at[g])
        pltpu.async_copy(h2_hbm.at[nids], rows, sem).wait()
        pltpu.sync_copy(rows.at[pl.ds(0, K)], d2_hbm.at[g])
        pltpu.async_copy(v_hbm.at[nids], vals, sem).wait()
        pltpu.sync_copy(vals, dv_hbm.at[g])
        return carry

    lax.fori_loop(0, 8, per_graph, 0)


_sc_sortpool = functools.partial(
    pl.kernel,
    _sortpool_body,
    out_type=(
        jax.ShapeDtypeStruct((B_GRAPHS, K, 32), jnp.float32),
        jax.ShapeDtypeStruct((B_GRAPHS, K, 32), jnp.float32),
        jax.ShapeDtypeStruct((B_GRAPHS, K, 32), jnp.float32),
        jax.ShapeDtypeStruct((B_GRAPHS, 32), jnp.float32),
    ),
    mesh=_MESH,
    compiler_params=_CP,
    scratch_types=[
        pltpu.VMEM((NACC,), jnp.float32),
        pltpu.VMEM((272,), jnp.int32),
        pltpu.VMEM((272,), jnp.int32),
        pltpu.VMEM((32,), jnp.int32),
        pltpu.VMEM((32, 32), jnp.float32),
        pltpu.VMEM((32,), jnp.float32),
        pltpu.SemaphoreType.DMA,
    ],
)()


def _head_body(h_ref, w1_ref, b1_ref, w2_ref, b2_ref, o_ref):
    h = h_ref[...]
    a = jnp.maximum(
        jnp.dot(h, w1_ref[...], preferred_element_type=jnp.float32)
        + b1_ref[...][None, :], 0.0)
    o_ref[...] = jnp.dot(a, w2_ref[...],
                         preferred_element_type=jnp.float32) + b2_ref[...][None, :]


def kernel(num_nodes, z, edge_index, batch, z_table, W0, b0, W1, b1, W2, b2,
           W3, b3, c1w, c1b, c2w, c2b, m1w, m1b, m2w, m2b):
    n = z.shape[0]
    pad = E_PAD - E
    src_p = jnp.concatenate([edge_index[0], jnp.zeros((pad,), jnp.int32)])
    dst_p = jnp.concatenate([edge_index[1], jnp.full((pad,), N, jnp.int32)])
    src2d = src_p.reshape(ROWS, 128)
    srcoff = jnp.stack([src2d, src2d + N])
    dst2d = dst_p.reshape(ROWS, 128)
    zeros1 = jnp.zeros((STRIPE,), jnp.float32)

    deg_p = _sc_scal(jnp.ones((N,), jnp.float32), srcoff, dst2d, zeros1)
    deg = deg_p[0, :N] + deg_p[1, :N] + 1.0
    dinv = jnp.where(deg > 0, deg ** -0.5, 0.0)
    dinv_pad = jnp.concatenate([dinv, jnp.zeros((NACC - N,), jnp.float32)])
    norm2d = _sc_norm(dinv_pad, srcoff, dst2d)
    nself = (dinv * dinv)[:, None]

    dst_all = jnp.concatenate([edge_index[1], jnp.arange(n)])

    x = z_table[z]
    hs = []
    h_prev = x
    for (W, b) in [(W0, b0), (W1, b1), (W2, b2)]:
        q = h_prev @ W
        qstack = jnp.concatenate([q[:, :16], q[:, 16:]], axis=0)
        rowse = _sc_upd(qstack, norm2d, srcoff)
        upd_e = jnp.concatenate(
            [rowse[0].reshape(E_PAD, 16)[:E], rowse[1].reshape(E_PAD, 16)[:E]],
            axis=1)
        upd = jnp.concatenate([upd_e, q * nself], axis=0)
        out = jnp.zeros((n, 32), jnp.float32).at[dst_all].add(upd)
        h_prev = jnp.tanh(out + b)
        hs.append(h_prev)

    q3 = h_prev @ W3
    norm_e = norm2d.reshape(E_PAD)[:E]
    upd3 = jnp.concatenate(
        [q3[edge_index[0], :] * norm_e[:, None], q3 * nself], axis=0)
    out3 = jnp.zeros((n, 1), jnp.float32).at[dst_all].add(upd3)
    v = jnp.tanh(out3 + b3)[:, 0]

    zpad = jnp.zeros((NACC - N, 32), jnp.float32)
    h0p = jnp.concatenate([hs[0], zpad])
    h1p = jnp.concatenate([hs[1], zpad])
    h2p = jnp.concatenate([hs[2], zpad])
    v_pad = jnp.concatenate([v, jnp.zeros((NACC - N,), jnp.float32)])
    starts = jnp.searchsorted(batch, jnp.arange(257)).astype(jnp.int32)
    starts_p = jnp.concatenate([starts[:256], jnp.full((16,), N, jnp.int32)])
    ends_p = jnp.concatenate([starts[1:257], jnp.full((16,), N, jnp.int32)])
    d0, d1, d2, dv = _sc_sortpool(v_pad, starts_p, ends_p, h0p, h1p, h2p)
    dense = jnp.concatenate([d0, d1, d2, dv[:, :K, None]], axis=-1)

    D = 97
    xf = dense.reshape(B_GRAPHS, 1, K * D)
    h = jax.nn.relu(lax.conv_general_dilated(
        xf, c1w, (D,), 'VALID', dimension_numbers=('NCH', 'OIH', 'NCH'))
        + c1b[None, :, None])
    h = lax.reduce_window(h, -jnp.inf, lax.max, (1, 1, 2), (1, 1, 2), 'VALID')
    h = jax.nn.relu(lax.conv_general_dilated(
        h, c2w, (1,), 'VALID', dimension_numbers=('NCH', 'OIH', 'NCH'))
        + c2b[None, :, None])
    h = h.reshape(B_GRAPHS, -1)
    out = pl.pallas_call(
        _head_body,
        out_shape=jax.ShapeDtypeStruct((B_GRAPHS, 1), jnp.float32),
    )(h, m1w, m1b, m2w, m2b)
    return out


# speedup vs baseline: 1.7451x; 1.0163x over previous
import functools

import jax
import jax.numpy as jnp
from jax import lax
from jax.experimental import pallas as pl
from jax.experimental.pallas import tpu as pltpu
from jax.experimental.pallas import tpu_sc as plsc

N = 100000
E = 1600000
ROWS = 12544
E_PAD = ROWS * 128
NACC = 100352
STRIPE = NACC // 16
B_GRAPHS = 256
K = 30

_MESH = plsc.VectorSubcoreMesh(core_axis_name="c", subcore_axis_name="s")
_CP = pltpu.CompilerParams(use_tc_tiling_on_sc=False, needs_layout_passes=False)


def _scal_body(v_hbm, srcoff_hbm, dst_hbm, zeros_hbm, out_hbm,
               idx_s, idx_d, vals, acc, sem):
    c = lax.axis_index("c")
    s = lax.axis_index("s")
    pltpu.sync_copy(zeros_hbm, acc.at[pl.ds(s * STRIPE, STRIPE)])
    plsc.subcore_barrier()

    def body(i, carry):
        base = (c * 16 + s) * 392 + i * 8
        pltpu.sync_copy(srcoff_hbm.at[0, pl.ds(base, 8)], idx_s)
        pltpu.sync_copy(dst_hbm.at[pl.ds(base, 8)], idx_d)
        ds = [pltpu.async_copy(v_hbm.at[idx_s.at[j]], vals.at[j], sem)
              for j in range(8)]
        for d in ds:
            d.wait()
        for j in range(8):
            pltpu.sync_copy(vals.at[j], acc.at[idx_d.at[j]], add=True)
        return carry

    lax.fori_loop(0, 49, body, 0)
    plsc.subcore_barrier()
    pltpu.sync_copy(acc.at[pl.ds(s * STRIPE, STRIPE)],
                    out_hbm.at[c, pl.ds(s * STRIPE, STRIPE)])


_sc_scal = functools.partial(
    pl.kernel,
    _scal_body,
    out_type=jax.ShapeDtypeStruct((2, NACC), jnp.float32),
    mesh=_MESH,
    compiler_params=_CP,
    scratch_types=[
        pltpu.VMEM((8, 128), jnp.int32),
        pltpu.VMEM((8, 128), jnp.int32),
        pltpu.VMEM((8, 128), jnp.float32),
        pltpu.VMEM_SHARED((NACC,), jnp.float32),
        pltpu.SemaphoreType.DMA,
    ],
)()


def _norm_body(dinv_hbm, srcoff_hbm, dst_hbm, norm_hbm, dv, idx_s, idx_d, nb):
    c = lax.axis_index("c")
    s = lax.axis_index("s")
    pltpu.sync_copy(dinv_hbm, dv)

    def body(i, carry):
        base = (c * 16 + s) * 392 + i * 8
        pltpu.sync_copy(srcoff_hbm.at[0, pl.ds(base, 8)], idx_s)
        pltpu.sync_copy(dst_hbm.at[pl.ds(base, 8)], idx_d)
        for j in range(8):
            for g in range(8):
                s16 = idx_s[j, pl.ds(g * 16, 16)]
                d16 = idx_d[j, pl.ds(g * 16, 16)]
                a = plsc.load_gather(dv, [s16])
                b = plsc.load_gather(dv, [d16])
                nb[j, pl.ds(g * 16, 16)] = a * b
        pltpu.sync_copy(nb, norm_hbm.at[pl.ds(base, 8)])
        return carry

    lax.fori_loop(0, 49, body, 0)


_sc_norm = functools.partial(
    pl.kernel,
    _norm_body,
    out_type=jax.ShapeDtypeStruct((ROWS, 128), jnp.float32),
    mesh=_MESH,
    compiler_params=_CP,
    scratch_types=[
        pltpu.VMEM((NACC,), jnp.float32),
        pltpu.VMEM((8, 128), jnp.int32),
        pltpu.VMEM((8, 128), jnp.int32),
        pltpu.VMEM((8, 128), jnp.float32),
    ],
)()


def _upd_body(q_hbm, norm_hbm, srcoff_hbm, rowse_hbm,
              idx_s, nb, rows, sem):
    c = lax.axis_index("c")
    s = lax.axis_index("s")
    iota = lax.iota(jnp.int32, 16)

    def body(i, carry):
        base = s * 784 + i * 8
        pltpu.sync_copy(srcoff_hbm.at[c, pl.ds(base, 8)], idx_s)
        ds = [pltpu.async_copy(q_hbm.at[idx_s.at[j]], rows.at[j], sem)
              for j in range(8)]
        pltpu.sync_copy(norm_hbm.at[pl.ds(base, 8)], nb)
        for d in ds:
            d.wait()
        for j in range(8):
            jc = jnp.full((16,), j, jnp.int32)
            for g in range(8):
                n16 = nb[j, pl.ds(g * 16, 16)]
                e16 = g * 16 + iota
                for col in range(16):
                    c16 = jnp.full((16,), col, jnp.int32)
                    vv = plsc.load_gather(rows, [jc, e16, c16])
                    plsc.store_scatter(rows, [jc, e16, c16], vv * n16)
        pltpu.sync_copy(rows, rowse_hbm.at[c, pl.ds(base, 8)])
        return carry

    lax.fori_loop(0, 98, body, 0)


_sc_upd = functools.partial(
    pl.kernel,
    _upd_body,
    out_type=jax.ShapeDtypeStruct((2, ROWS, 128, 16), jnp.float32),
    mesh=_MESH,
    compiler_params=_CP,
    scratch_types=[
        pltpu.VMEM((8, 128), jnp.int32),
        pltpu.VMEM((8, 128), jnp.float32),
        pltpu.VMEM((8, 128, 16), jnp.float32),
        pltpu.SemaphoreType.DMA,
    ],
)()


def _sortpool_body(v_hbm, starts_hbm, ends_hbm, h0_hbm, h1_hbm, h2_hbm,
                   d0_hbm, d1_hbm, d2_hbm, dv_hbm,
                   vbuf, sbuf, ebuf, nids, rows, vals, sem):
    c = lax.axis_index("c")
    s = lax.axis_index("s")
    w = c * 16 + s
    pltpu.sync_copy(v_hbm, vbuf)
    pltpu.sync_copy(starts_hbm, sbuf)
    pltpu.sync_copy(ends_hbm, ebuf)
    iota = lax.iota(jnp.int32, 16)
    lane0 = iota == 0
    big = jnp.full((16,), 1 << 29, jnp.int32)

    def per_graph(gi, carry):
        g = w * 8 + gi
        off = (g // 16) * 16
        lane = g - off
        svec = jnp.where(iota == lane, sbuf[pl.ds(off, 16)], 0)
        evec = jnp.where(iota == lane, ebuf[pl.ds(off, 16)], 0)
        sg = jnp.max(svec)
        eg = jnp.max(evec)
        cs = sg // 16
        nch = (eg + 15) // 16 - cs
        cnt = eg - sg
        nids[pl.ds(0, 16)] = jnp.full((16,), N, jnp.int32)
        nids[pl.ds(16, 16)] = jnp.full((16,), N, jnp.int32)

        def per_round(r, carry2):
            def per_chunk(i, mp):
                m16, p16 = mp
                base = (cs + i) * 16
                val = vbuf[pl.ds(base, 16)]
                posv = base + iota
                ok = (posv >= sg) & (posv < eg)
                vv = jnp.where(ok, val, jnp.float32(-4.0))
                upd = vv > m16
                return (jnp.where(upd, vv, m16), jnp.where(upd, posv, p16))

            m16, p16 = lax.fori_loop(
                0, nch, per_chunk,
                (jnp.full((16,), -4.0, jnp.float32), big))
            M = jnp.max(m16)
            pm = jnp.where(m16 == M, p16, big)
            P = jnp.min(pm)
            psplat = jnp.full((16,), P, jnp.int32)
            plsc.store_scatter(nids, [jnp.full((16,), r, jnp.int32)],
                               psplat, mask=lane0)
            plsc.store_scatter(vbuf, [psplat],
                               jnp.full((16,), -4.0, jnp.float32), mask=lane0)
            return carry2

        lax.fori_loop(0, jnp.minimum(cnt, K), per_round, 0)

        pltpu.async_copy(h0_hbm.at[nids], rows, sem).wait()
        pltpu.sync_copy(rows.at[pl.ds(0, K)], d0_hbm.at[g])
        pltpu.async_copy(h1_hbm.at[nids], rows, sem).wait()
        pltpu.sync_copy(rows.at[pl.ds(0, K)], d1_hbm.at[g])
        pltpu.async_copy(h2_hbm.at[nids], rows, sem).wait()
        pltpu.sync_copy(rows.at[pl.ds(0, K)], d2_hbm.at[g])
        pltpu.async_copy(v_hbm.at[nids], vals, sem).wait()
        pltpu.sync_copy(vals, dv_hbm.at[g])
        return carry

    lax.fori_loop(0, 8, per_graph, 0)


_sc_sortpool = functools.partial(
    pl.kernel,
    _sortpool_body,
    out_type=(
        jax.ShapeDtypeStruct((B_GRAPHS, K, 32), jnp.float32),
        jax.ShapeDtypeStruct((B_GRAPHS, K, 32), jnp.float32),
        jax.ShapeDtypeStruct((B_GRAPHS, K, 32), jnp.float32),
        jax.ShapeDtypeStruct((B_GRAPHS, 32), jnp.float32),
    ),
    mesh=_MESH,
    compiler_params=_CP,
    scratch_types=[
        pltpu.VMEM((NACC,), jnp.float32),
        pltpu.VMEM((272,), jnp.int32),
        pltpu.VMEM((272,), jnp.int32),
        pltpu.VMEM((32,), jnp.int32),
        pltpu.VMEM((32, 32), jnp.float32),
        pltpu.VMEM((32,), jnp.float32),
        pltpu.SemaphoreType.DMA,
    ],
)()


def _head_body(h_ref, w1_ref, b1_ref, w2_ref, b2_ref, o_ref):
    h = h_ref[...]
    a = jnp.maximum(
        jnp.dot(h, w1_ref[...], preferred_element_type=jnp.float32)
        + b1_ref[...][None, :], 0.0)
    o_ref[...] = jnp.dot(a, w2_ref[...],
                         preferred_element_type=jnp.float32) + b2_ref[...][None, :]


def kernel(num_nodes, z, edge_index, batch, z_table, W0, b0, W1, b1, W2, b2,
           W3, b3, c1w, c1b, c2w, c2b, m1w, m1b, m2w, m2b):
    n = z.shape[0]
    pad = E_PAD - E
    src_p = jnp.concatenate([edge_index[0], jnp.zeros((pad,), jnp.int32)])
    dst_p = jnp.concatenate([edge_index[1], jnp.full((pad,), N, jnp.int32)])
    src2d = src_p.reshape(ROWS, 128)
    srcoff = jnp.stack([src2d, src2d + N])
    dst2d = dst_p.reshape(ROWS, 128)
    zeros1 = jnp.zeros((STRIPE,), jnp.float32)

    deg_p = _sc_scal(jnp.ones((N,), jnp.float32), srcoff, dst2d, zeros1)
    deg = deg_p[0, :N] + deg_p[1, :N] + 1.0
    dinv = jnp.where(deg > 0, deg ** -0.5, 0.0)
    dinv_pad = jnp.concatenate([dinv, jnp.zeros((NACC - N,), jnp.float32)])
    norm2d = _sc_norm(dinv_pad, srcoff, dst2d)
    nself = (dinv * dinv)[:, None]

    dst_all = jnp.concatenate([edge_index[1], jnp.arange(n)])

    x = z_table[z]
    hs = []
    h_prev = x
    for (W, b) in [(W0, b0), (W1, b1), (W2, b2)]:
        q = h_prev @ W
        qstack = jnp.concatenate([q[:, :16], q[:, 16:]], axis=0)
        rowse = _sc_upd(qstack, norm2d, srcoff)
        upd_e = jnp.concatenate(
            [rowse[0].reshape(E_PAD, 16)[:E], rowse[1].reshape(E_PAD, 16)[:E]],
            axis=1)
        upd = jnp.concatenate([upd_e, q * nself], axis=0)
        out = jnp.zeros((n, 32), jnp.float32).at[dst_all].add(upd)
        h_prev = jnp.tanh(out + b)
        hs.append(h_prev)

    q3 = h_prev @ W3
    norm_e = norm2d.reshape(E_PAD)[:E]
    upd3 = jnp.concatenate(
        [q3[edge_index[0], :] * norm_e[:, None], q3 * nself], axis=0)
    out3 = jnp.zeros((n, 1), jnp.float32).at[dst_all].add(upd3)
    v = jnp.tanh(out3 + b3)[:, 0]

    zpad = jnp.zeros((NACC - N, 32), jnp.float32)
    h0p = jnp.concatenate([hs[0], zpad])
    h1p = jnp.concatenate([hs[1], zpad])
    h2p = jnp.concatenate([hs[2], zpad])
    v_pad = jnp.concatenate([v, jnp.zeros((NACC - N,), jnp.float32)])
    starts = jnp.searchsorted(batch, jnp.arange(257)).astype(jnp.int32)
    starts_p = jnp.concatenate([starts[:256], jnp.full((16,), N, jnp.int32)])
    ends_p = jnp.concatenate([starts[1:257], jnp.full((16,), N, jnp.int32)])
    d0, d1, d2, dv = _sc_sortpool(v_pad, starts_p, ends_p, h0p, h1p, h2p)
    dense = jnp.concatenate([d0, d1, d2, dv[:, :K, None]], axis=-1)

    D = 97
    xf = dense.reshape(B_GRAPHS, 1, K * D)
    h = jax.nn.relu(lax.conv_general_dilated(
        xf, c1w, (D,), 'VALID', dimension_numbers=('NCH', 'OIH', 'NCH'))
        + c1b[None, :, None])
    h = lax.reduce_window(h, -jnp.inf, lax.max, (1, 1, 2), (1, 1, 2), 'VALID')
    h = jax.nn.relu(lax.conv_general_dilated(
        h, c2w, (1,), 'VALID', dimension_numbers=('NCH', 'OIH', 'NCH'))
        + c2b[None, :, None])
    h = h.reshape(B_GRAPHS, -1)
    out = pl.pallas_call(
        _head_body,
        out_shape=jax.ShapeDtypeStruct((B_GRAPHS, 1), jnp.float32),
    )(h, m1w, m1b, m2w, m2b)
    return out
